# baseline (speedup 1.0000x reference)
import functools

import jax
import jax.numpy as jnp
from jax import lax
from jax.experimental import pallas as pl
from jax.experimental.pallas import tpu as pltpu
from jax.experimental.pallas import tpu_sc as plsc

NU = 5000
NI = 5000
EMB = 256
NL = 5
NE = 80000
NB = 4096
TPS = 16
NSC = 2
PSIDE = 5120
NP = 2 * PSIDE
RPT = PSIDE // TPS
CH = 64
NCH = 80
EPT = NCH * CH
EC = 128
NBUF = 3
LAG = NBUF - 1
WB = CH
VL = 16
BPW = NB // (NSC * TPS)

_mesh = plsc.VectorSubcoreMesh(core_axis_name="c", subcore_axis_name="s")
_f32 = jnp.float32


def _deg_body(rowsq, deg_out, deg_sh, idxv, onesv, stg):
  cid = lax.axis_index("c")
  sid = lax.axis_index("s")

  @pl.loop(0, RPT // VL)
  def _(i):
    stg[pl.ds(i * VL, VL)] = jnp.zeros((VL,), _f32)

  pltpu.sync_copy(stg, deg_sh.at[pl.ds(sid * RPT, RPT)])

  @pl.loop(0, CH // VL)
  def _(i):
    onesv[pl.ds(i * VL, VL)] = jnp.ones((VL,), _f32)

  pltpu.sync_copy(rowsq.at[cid, sid], idxv)
  plsc.subcore_barrier()

  @pl.loop(0, NCH)
  def _(j):
    pltpu.sync_copy(onesv, deg_sh.at[idxv.at[j]], add=True)

  plsc.subcore_barrier()
  pltpu.sync_copy(deg_sh.at[pl.ds(sid * RPT, RPT)], stg)
  pltpu.sync_copy(stg, deg_out.at[pl.ds(cid * PSIDE + sid * RPT, RPT)])


_deg_call = pl.kernel(
    _deg_body,
    out_type=jax.ShapeDtypeStruct((NP,), _f32),
    mesh=_mesh,
    scratch_types=[
        pltpu.VMEM_SHARED((PSIDE,), _f32),
        pltpu.VMEM((NCH, CH), jnp.int32),
        pltpu.VMEM((CH,), _f32),
        pltpu.VMEM((RPT,), _f32),
    ],
)


def _prep_body(deg_ref, emb_ref, dinv_ref, dinv2_ref, z0_ref, z1_ref):
  d = deg_ref[...]
  dinv = jnp.where(d > 0.0, lax.rsqrt(jnp.maximum(d, 1e-30)), 0.0)
  dinv_ref[...] = dinv
  dinv2_ref[...] = dinv * dinv
  z0_ref[...] = emb_ref[:, :EC] * dinv
  z1_ref[...] = emb_ref[:, EC:] * dinv


_prep_call = pl.pallas_call(
    _prep_body,
    out_shape=(
        jax.ShapeDtypeStruct((NP, 1), _f32),
        jax.ShapeDtypeStruct((NP, 1), _f32),
        jax.ShapeDtypeStruct((NP, EC), _f32),
        jax.ShapeDtypeStruct((NP, EC), _f32),
    ),
)


def _layer_body(zin, rowsq, colsq, so,
                acc_sh, z_sh, rv, cv, *gbs):
  gbufs = gbs[:NBUF]
  gsems = gbs[NBUF:2 * NBUF]
  ssems = gbs[2 * NBUF:]
  sbuf = gbufs[0]
  cid = lax.axis_index("c")
  sid = lax.axis_index("s")
  gbase = cid * PSIDE + sid * RPT
  lbase = sid * RPT
  sbase = (1 - cid) * PSIDE + sid * RPT

  pltpu.sync_copy(rowsq.at[cid, sid], rv)
  pltpu.sync_copy(colsq.at[cid, sid], cv)

  if True:
    pltpu.sync_copy(zin.at[pl.ds(sbase, RPT)], z_sh.at[pl.ds(lbase, RPT)])

    @pl.loop(0, WB)
    def _(r):
      for v in range(EC // VL):
        sbuf[r, pl.ds(v * VL, VL)] = jnp.zeros((VL,), _f32)

    for k in range(RPT // WB):
      pltpu.sync_copy(sbuf, acc_sh.at[pl.ds(lbase + k * WB, WB)])
    plsc.subcore_barrier()

    gh = [None] * NBUF
    sh = [None] * NBUF
    for j in range(NCH + LAG):
      if j < NCH:
        b = j % NBUF
        if sh[b] is not None:
          sh[b].wait()
          sh[b] = None
        gh[b] = pltpu.async_copy(z_sh.at[cv.at[j]], gbufs[b], gsems[b])
      if j >= LAG:
        jj = j - LAG
        b = jj % NBUF
        gh[b].wait()
        gh[b] = None
        sh[b] = pltpu.async_copy(gbufs[b], acc_sh.at[rv.at[jj]], ssems[b],
                                 add=True)
    for b in range(NBUF):
      if sh[b] is not None:
        sh[b].wait()

    plsc.subcore_barrier()

    pltpu.sync_copy(acc_sh.at[pl.ds(lbase, RPT)], so.at[pl.ds(gbase, RPT)])


_layer_call = pl.kernel(
    _layer_body,
    out_type=jax.ShapeDtypeStruct((NP, EC), _f32),
    mesh=_mesh,
    scratch_types=[
        pltpu.VMEM_SHARED((PSIDE, EC), _f32),
        pltpu.VMEM_SHARED((PSIDE, EC), _f32),
        pltpu.VMEM((NCH, CH), jnp.int32),
        pltpu.VMEM((NCH, CH), jnp.int32),
    ] + [pltpu.VMEM((CH, EC), _f32)] * NBUF
      + [pltpu.SemaphoreType.DMA] * (2 * NBUF),
)


def _scale_body(s, acc, dinv, dinv2, zo, acco):
  di = dinv[...]
  di2 = dinv2[...]
  a = s[...]
  zo[...] = di2 * a
  acco[...] = acc[...] + di * a


_scale_call = pl.pallas_call(
    _scale_body,
    out_shape=(
        jax.ShapeDtypeStruct((NP, EC), _f32),
        jax.ShapeDtypeStruct((NP, EC), _f32),
    ),
)


def _out_body(acc0, acc1, uidx, iidx, u0, u1, i0, i1, idxv, buf, sem):
  cid = lax.axis_index("c")
  sid = lax.axis_index("s")
  base = (cid * TPS + sid) * BPW

  for src, dsts in ((uidx, (u0, u1)), (iidx, (i0, i1))):
    pltpu.sync_copy(src.at[pl.ds(base, BPW)], idxv)
    for acch, dst in ((acc0, dsts[0]), (acc1, dsts[1])):
      pltpu.async_copy(acch.at[idxv], buf, sem).wait()

      @pl.loop(0, BPW)
      def _(r):
        for v in range(EC // VL):
          buf[r, pl.ds(v * VL, VL)] = buf[r, pl.ds(v * VL, VL)] * (1.0 / 6.0)

      pltpu.sync_copy(buf, dst.at[pl.ds(base, BPW)])


_out_call = pl.kernel(
    _out_body,
    out_type=tuple(jax.ShapeDtypeStruct((NB, EC), _f32) for _ in range(4)),
    mesh=_mesh,
    scratch_types=[
        pltpu.VMEM((BPW,), jnp.int32),
        pltpu.VMEM((BPW, EC), _f32),
        pltpu.SemaphoreType.DMA,
    ],
)


def _pack_side(r_local, c_local):
  npad = TPS * EPT - NE
  r = jnp.concatenate([r_local, jnp.full((npad,), PSIDE - 1, jnp.int32)])
  c = jnp.concatenate([c_local, jnp.full((npad,), PSIDE - 1, jnp.int32)])
  return r.reshape(TPS, NCH, CH), c.reshape(TPS, NCH, CH)


def kernel(users, items, user_table, item_table, rows, cols, vals):
  users = users.astype(jnp.int32)
  items = items.astype(jnp.int32)
  rows = rows.astype(jnp.int32)
  cols = cols.astype(jnp.int32)

  emb = jnp.concatenate([
      jnp.pad(user_table.astype(_f32), ((0, PSIDE - NU), (0, 0))),
      jnp.pad(item_table.astype(_f32), ((0, PSIDE - NI), (0, 0))),
  ], axis=0)

  r0, c0 = _pack_side(rows[:NE], cols[:NE] - NU)
  r1, c1 = _pack_side(rows[NE:] - NU, cols[NE:])
  rowsq = jnp.stack([r0, r1])
  colsq = jnp.stack([c0, c1])

  deg = _deg_call(rowsq).reshape(NP, 1)
  dinv, dinv2, z0h, z1h = _prep_call(deg, emb)

  acc0 = emb[:, :EC]
  acc1 = emb[:, EC:]
  for _ in range(NL):
    s0 = _layer_call(z0h, rowsq, colsq)
    s1 = _layer_call(z1h, rowsq, colsq)
    z0h, acc0 = _scale_call(s0, acc0, dinv, dinv2)
    z1h, acc1 = _scale_call(s1, acc1, dinv, dinv2)

  u0, u1, i0, i1 = _out_call(acc0, acc1, users, items + PSIDE)
  return (jnp.concatenate([u0, u1], axis=1),
          jnp.concatenate([i0, i1], axis=1))


# speedup vs baseline: 1.0295x; 1.0295x over previous
import functools

import jax
import jax.numpy as jnp
from jax import lax
from jax.experimental import pallas as pl
from jax.experimental.pallas import tpu as pltpu
from jax.experimental.pallas import tpu_sc as plsc

NU = 5000
NI = 5000
EMB = 256
NL = 5
NE = 80000
NB = 4096
TPS = 16
NSC = 2
PSIDE = 5120
NP = 2 * PSIDE
RPT = PSIDE // TPS
CH = 64
NCH = 80
EPT = NCH * CH
EC = 128
NBUF = 3
LAG = NBUF - 1
WB = CH
VL = 16
BPW = NB // (NSC * TPS)

_mesh = plsc.VectorSubcoreMesh(core_axis_name="c", subcore_axis_name="s")
_f32 = jnp.float32


def _deg_body(rowsq, deg_out, deg_sh, idxv, onesv, stg):
  cid = lax.axis_index("c")
  sid = lax.axis_index("s")

  @pl.loop(0, RPT // VL)
  def _(i):
    stg[pl.ds(i * VL, VL)] = jnp.zeros((VL,), _f32)

  pltpu.sync_copy(stg, deg_sh.at[pl.ds(sid * RPT, RPT)])

  @pl.loop(0, CH // VL)
  def _(i):
    onesv[pl.ds(i * VL, VL)] = jnp.ones((VL,), _f32)

  pltpu.sync_copy(rowsq.at[cid, sid], idxv)
  plsc.subcore_barrier()

  @pl.loop(0, NCH)
  def _(j):
    pltpu.sync_copy(onesv, deg_sh.at[idxv.at[j]], add=True)

  plsc.subcore_barrier()
  pltpu.sync_copy(deg_sh.at[pl.ds(sid * RPT, RPT)], stg)
  pltpu.sync_copy(stg, deg_out.at[pl.ds(cid * PSIDE + sid * RPT, RPT)])


_deg_call = pl.kernel(
    _deg_body,
    out_type=jax.ShapeDtypeStruct((NP,), _f32),
    mesh=_mesh,
    scratch_types=[
        pltpu.VMEM_SHARED((PSIDE,), _f32),
        pltpu.VMEM((NCH, CH), jnp.int32),
        pltpu.VMEM((CH,), _f32),
        pltpu.VMEM((RPT,), _f32),
    ],
)


def _prep_body(deg_ref, emb_ref, dinv_ref, dinv2_ref, z0_ref, z1_ref):
  d = deg_ref[...]
  dinv = jnp.where(d > 0.0, lax.rsqrt(jnp.maximum(d, 1e-30)), 0.0)
  dinv_ref[...] = dinv
  dinv2_ref[...] = dinv * dinv
  z0_ref[...] = emb_ref[:, :EC] * dinv
  z1_ref[...] = emb_ref[:, EC:] * dinv


_prep_call = pl.pallas_call(
    _prep_body,
    out_shape=(
        jax.ShapeDtypeStruct((NP, 1), _f32),
        jax.ShapeDtypeStruct((NP, 1), _f32),
        jax.ShapeDtypeStruct((NP, EC), _f32),
        jax.ShapeDtypeStruct((NP, EC), _f32),
    ),
)


def _layer_body(z0, z1, rowsq, colsq, s0out, s1out,
                acc_sh, z_sh, rv, cv, *gbs):
  gbufs = gbs[:NBUF]
  gsems = gbs[NBUF:2 * NBUF]
  ssems = gbs[2 * NBUF:3 * NBUF]
  wbsem = gbs[3 * NBUF]
  sbuf = gbufs[0]
  cid = lax.axis_index("c")
  sid = lax.axis_index("s")
  gbase = cid * PSIDE + sid * RPT
  lbase = sid * RPT
  sbase = (1 - cid) * PSIDE + sid * RPT

  pltpu.sync_copy(rowsq.at[cid, sid], rv)
  pltpu.sync_copy(colsq.at[cid, sid], cv)

  wbh = None
  for h, zin, so in ((0, z0, s0out), (1, z1, s1out)):
    pltpu.sync_copy(zin.at[pl.ds(sbase, RPT)], z_sh.at[pl.ds(lbase, RPT)])

    @pl.loop(0, WB)
    def _(r):
      for v in range(EC // VL):
        sbuf[r, pl.ds(v * VL, VL)] = jnp.zeros((VL,), _f32)

    if wbh is not None:
      wbh.wait()
      wbh = None
    for k in range(RPT // WB):
      pltpu.sync_copy(sbuf, acc_sh.at[pl.ds(lbase + k * WB, WB)])
    plsc.subcore_barrier()

    gh = [None] * NBUF
    sh = [None] * NBUF
    for j in range(NCH + LAG):
      if j < NCH:
        b = j % NBUF
        if sh[b] is not None:
          sh[b].wait()
          sh[b] = None
        gh[b] = pltpu.async_copy(z_sh.at[cv.at[j]], gbufs[b], gsems[b])
      if j >= LAG:
        jj = j - LAG
        b = jj % NBUF
        gh[b].wait()
        gh[b] = None
        sh[b] = pltpu.async_copy(gbufs[b], acc_sh.at[rv.at[jj]], ssems[b],
                                 add=True)
    for b in range(NBUF):
      if sh[b] is not None:
        sh[b].wait()

    plsc.subcore_barrier()

    wbh = pltpu.async_copy(acc_sh.at[pl.ds(lbase, RPT)],
                           so.at[pl.ds(gbase, RPT)], wbsem)
  wbh.wait()


_layer_call = pl.kernel(
    _layer_body,
    out_type=(
        jax.ShapeDtypeStruct((NP, EC), _f32),
        jax.ShapeDtypeStruct((NP, EC), _f32),
    ),
    mesh=_mesh,
    scratch_types=[
        pltpu.VMEM_SHARED((PSIDE, EC), _f32),
        pltpu.VMEM_SHARED((PSIDE, EC), _f32),
        pltpu.VMEM((NCH, CH), jnp.int32),
        pltpu.VMEM((NCH, CH), jnp.int32),
    ] + [pltpu.VMEM((CH, EC), _f32)] * NBUF
      + [pltpu.SemaphoreType.DMA] * (2 * NBUF + 1),
)


def _scale_body(s0, s1, acc, dinv, dinv2, z0o, z1o, acco):
  di = dinv[...]
  di2 = dinv2[...]
  a = s0[...]
  b = s1[...]
  z0o[...] = di2 * a
  z1o[...] = di2 * b
  acco[...] = acc[...] + jnp.concatenate([di * a, di * b], axis=1)


_scale_call = pl.pallas_call(
    _scale_body,
    out_shape=(
        jax.ShapeDtypeStruct((NP, EC), _f32),
        jax.ShapeDtypeStruct((NP, EC), _f32),
        jax.ShapeDtypeStruct((NP, EMB), _f32),
    ),
)


def _out_body(acc, uidx, iidx, uout, iout, idxv, buf, sem):
  cid = lax.axis_index("c")
  sid = lax.axis_index("s")
  base = (cid * TPS + sid) * BPW

  for src, dst in ((uidx, uout), (iidx, iout)):
    pltpu.sync_copy(src.at[pl.ds(base, BPW)], idxv)
    pltpu.async_copy(acc.at[idxv], buf, sem).wait()

    @pl.loop(0, BPW)
    def _(r):
      for v in range(EMB // VL):
        buf[r, pl.ds(v * VL, VL)] = buf[r, pl.ds(v * VL, VL)] * (1.0 / 6.0)

    pltpu.sync_copy(buf, dst.at[pl.ds(base, BPW)])


_out_call = pl.kernel(
    _out_body,
    out_type=(
        jax.ShapeDtypeStruct((NB, EMB), _f32),
        jax.ShapeDtypeStruct((NB, EMB), _f32),
    ),
    mesh=_mesh,
    scratch_types=[
        pltpu.VMEM((BPW,), jnp.int32),
        pltpu.VMEM((BPW, EMB), _f32),
        pltpu.SemaphoreType.DMA,
    ],
)


def _pack_side(r_local, c_local):
  npad = TPS * EPT - NE
  r = jnp.concatenate([r_local, jnp.full((npad,), PSIDE - 1, jnp.int32)])
  c = jnp.concatenate([c_local, jnp.full((npad,), PSIDE - 1, jnp.int32)])
  return r.reshape(TPS, NCH, CH), c.reshape(TPS, NCH, CH)


def kernel(users, items, user_table, item_table, rows, cols, vals):
  users = users.astype(jnp.int32)
  items = items.astype(jnp.int32)
  rows = rows.astype(jnp.int32)
  cols = cols.astype(jnp.int32)

  emb = jnp.concatenate([
      jnp.pad(user_table.astype(_f32), ((0, PSIDE - NU), (0, 0))),
      jnp.pad(item_table.astype(_f32), ((0, PSIDE - NI), (0, 0))),
  ], axis=0)

  r0, c0 = _pack_side(rows[:NE], cols[:NE] - NU)
  r1, c1 = _pack_side(rows[NE:] - NU, cols[NE:])
  rowsq = jnp.stack([r0, r1])
  colsq = jnp.stack([c0, c1])

  deg = _deg_call(rowsq).reshape(NP, 1)
  dinv, dinv2, z0h, z1h = _prep_call(deg, emb)

  acc = emb
  for _ in range(NL):
    s0, s1 = _layer_call(z0h, z1h, rowsq, colsq)
    z0h, z1h, acc = _scale_call(s0, s1, acc, dinv, dinv2)

  return _out_call(acc, users, items + PSIDE)
